# baseline (speedup 1.0000x reference)
import functools
import math

import jax
import jax.numpy as jnp
import numpy as np
from jax import lax
from jax.experimental import pallas as pl
from jax.experimental.pallas import tpu as pltpu
from jax.experimental.pallas import tpu_sc as plsc

_EMB = 64
_LEN = 200
_L_BLK = 8
_ROWS_PER_WORKER = 8
_N_ACTIVE = _LEN // _ROWS_PER_WORKER


def _make_pe_scaled():
    pe = np.zeros((_LEN, _EMB), dtype=np.float32)
    position = np.arange(0, _LEN).astype(np.float32)[:, None]
    div_term = np.exp(
        np.arange(0, _EMB, 2).astype(np.float32) * -(math.log(10000.0) / _EMB)
    )
    pe[:, 0::2] = np.sin(position * div_term)
    pe[:, 1::2] = np.cos(position * div_term)
    return jnp.asarray(0.01 * pe)[:, :, None]


def _decay_sc_body(seq_hbm, last_hbm, out_hbm, seq_v, last_v, out_v):
    info = plsc.get_sparse_core_info()
    nc = info.num_cores
    wid = lax.axis_index("s") * nc + lax.axis_index("c")

    @pl.when(wid < _N_ACTIVE)
    def _():
        base = wid * _ROWS_PER_WORKER
        pltpu.sync_copy(seq_hbm.at[pl.ds(base, _ROWS_PER_WORKER)], seq_v)
        pltpu.sync_copy(last_hbm, last_v)

        def row_loop(r, _):
            def vec_loop(j, _):
                s = seq_v[r, pl.ds(j * 16, 16)]
                t = last_v[pl.ds(j * 16, 16)]
                absd = jnp.abs(t - s).astype(jnp.float32)
                out_v[r, pl.ds(j * 16, 16)] = 1.0 / (
                    math.e + absd * (0.5 / 86400.0)
                )
                return 0

            return lax.fori_loop(0, 4096 // 16, vec_loop, 0)

        lax.fori_loop(0, _ROWS_PER_WORKER, row_loop, 0)
        pltpu.sync_copy(out_v, out_hbm.at[pl.ds(base, _ROWS_PER_WORKER)])


def _decay_on_sc(seq_t, last):
    mesh = plsc.VectorSubcoreMesh(core_axis_name="c", subcore_axis_name="s")
    batch = seq_t.shape[1]
    return pl.kernel(
        _decay_sc_body,
        out_type=jax.ShapeDtypeStruct((_LEN, batch), jnp.float32),
        mesh=mesh,
        scratch_types=[
            pltpu.VMEM((_ROWS_PER_WORKER, batch), jnp.int32),
            pltpu.VMEM((batch,), jnp.int32),
            pltpu.VMEM((_ROWS_PER_WORKER, batch), jnp.float32),
        ],
    )(seq_t, last)


def _fused_tc_kernel(emb_ref, decay_ref, pe_ref, out_ref):
    out_ref[...] = emb_ref[...] + decay_ref[...][:, None, :] + pe_ref[...]


def kernel(item_seq_emb, batch_seqs_item, batch_last_time, pos_table):
    del pos_table
    batch = item_seq_emb.shape[0]
    emb_t = jnp.transpose(item_seq_emb, (1, 2, 0))
    seq_t = jnp.transpose(batch_seqs_item, (1, 0))
    pe_scaled = _make_pe_scaled()
    decay_t = _decay_on_sc(seq_t, batch_last_time)
    out_t = pl.pallas_call(
        _fused_tc_kernel,
        grid=(_LEN // _L_BLK,),
        in_specs=[
            pl.BlockSpec((_L_BLK, _EMB, batch), lambda i: (i, 0, 0)),
            pl.BlockSpec((_L_BLK, batch), lambda i: (i, 0)),
            pl.BlockSpec((_L_BLK, _EMB, 1), lambda i: (i, 0, 0)),
        ],
        out_specs=pl.BlockSpec((_L_BLK, _EMB, batch), lambda i: (i, 0, 0)),
        out_shape=jax.ShapeDtypeStruct((_LEN, _EMB, batch), jnp.float32),
    )(emb_t, decay_t, pe_scaled)
    return jnp.transpose(out_t, (2, 0, 1))


# speedup vs baseline: 1.4028x; 1.4028x over previous
import math

import jax
import jax.numpy as jnp
import numpy as np
from jax.experimental import pallas as pl

_EMB = 64
_LEN = 200
_L_BLK = 8


def _make_pe_scaled():
    pe = np.zeros((_LEN, _EMB), dtype=np.float32)
    position = np.arange(0, _LEN).astype(np.float32)[:, None]
    div_term = np.exp(
        np.arange(0, _EMB, 2).astype(np.float32) * -(math.log(10000.0) / _EMB)
    )
    pe[:, 0::2] = np.sin(position * div_term)
    pe[:, 1::2] = np.cos(position * div_term)
    return jnp.asarray(0.01 * pe)[:, :, None]


def _fused_kernel(emb_ref, seq_ref, last_ref, pe_ref, out_ref):
    diff = last_ref[0:1, :] - seq_ref[...]
    absd = jnp.abs(diff).astype(jnp.float32)
    decay = 1.0 / (math.e + absd * (0.5 / 86400.0))
    out_ref[...] = emb_ref[...] + decay[:, None, :] + pe_ref[...]


def kernel(item_seq_emb, batch_seqs_item, batch_last_time, pos_table):
    del pos_table
    batch = item_seq_emb.shape[0]
    emb_t = jnp.transpose(item_seq_emb, (1, 2, 0))
    seq_t = jnp.transpose(batch_seqs_item, (1, 0))
    last_row = batch_last_time[None, :]
    pe_scaled = _make_pe_scaled()
    grid = (_LEN // _L_BLK,)
    out_t = pl.pallas_call(
        _fused_kernel,
        grid=grid,
        in_specs=[
            pl.BlockSpec((_L_BLK, _EMB, batch), lambda i: (i, 0, 0)),
            pl.BlockSpec((_L_BLK, batch), lambda i: (i, 0)),
            pl.BlockSpec((1, batch), lambda i: (0, 0)),
            pl.BlockSpec((_L_BLK, _EMB, 1), lambda i: (i, 0, 0)),
        ],
        out_specs=pl.BlockSpec((_L_BLK, _EMB, batch), lambda i: (i, 0, 0)),
        out_shape=jax.ShapeDtypeStruct((_LEN, _EMB, batch), jnp.float32),
    )(emb_t, seq_t, last_row, pe_scaled)
    return jnp.transpose(out_t, (2, 0, 1))
